# Initial kernel scaffold; baseline (speedup 1.0000x reference)
import jax
import jax.numpy as jnp
from jax.experimental import pallas as pl


def kernel(x, segment_ids, num_segments, W, b):
    raise NotImplementedError("write your pallas kernel here")


# speedup vs baseline: 7.3774x; 7.3774x over previous
import jax
import jax.numpy as jnp
from jax import lax
from jax.experimental import pallas as pl
from jax.experimental.pallas import tpu as pltpu
from jax.experimental.pallas import tpu_sc as plsc

_N = 320000
_D = 128
_S = 10000

_BN = 2000
_RPT = _N // 32
_SUB = 80
_NSUB = _RPT // _SUB
_SP = 10240
_SLICE = _SP // 16


def _weights_body(x_ref, w_ref, b_ref, y_ref, e_ref):
    xb = x_ref[...]
    logit = jnp.dot(xb, w_ref[...], preferred_element_type=jnp.float32)
    e = jnp.exp(logit + b_ref[0, 0])
    y_ref[...] = xb * e
    e_ref[...] = e


def _weights(x, W, b2):
    return pl.pallas_call(
        _weights_body,
        grid=(_N // _BN,),
        in_specs=[
            pl.BlockSpec((_BN, _D), lambda i: (i, 0)),
            pl.BlockSpec((_D, 1), lambda i: (0, 0)),
            pl.BlockSpec((1, 1), lambda i: (0, 0)),
        ],
        out_specs=[
            pl.BlockSpec((_BN, _D), lambda i: (i, 0)),
            pl.BlockSpec((_BN, 1), lambda i: (i, 0)),
        ],
        out_shape=[
            jax.ShapeDtypeStruct((_N, _D), jnp.float32),
            jax.ShapeDtypeStruct((_N, 1), jnp.float32),
        ],
    )(x, W, b2)


def _scatter_body(y_hbm, e_hbm, seg_hbm, numer_out, den_out,
                  nsh, dsh, ybuf, segbuf, ebuf, zv):
    c = lax.axis_index("c")
    s = lax.axis_index("s")
    tile = c * 16 + s
    row0 = tile * _RPT

    z16 = jnp.zeros((16,), jnp.float32)

    def zy(i, _):
        for j in range(8):
            ybuf[i, pl.ds(16 * j, 16)] = z16
        return 0

    lax.fori_loop(0, _SUB, zy, 0)

    for q in range(_SLICE // 16):
        zv[pl.ds(16 * q, 16)] = z16

    base = pl.multiple_of(s * _SLICE, 8)
    for j in range(_SLICE // _SUB):
        pltpu.sync_copy(ybuf, nsh.at[pl.ds(base + _SUB * j, _SUB)])
    pltpu.sync_copy(zv, dsh.at[pl.ds(base, _SLICE)])
    plsc.subcore_barrier()

    def sub(j, _):
        r0 = pl.multiple_of(row0 + j * _SUB, 8)
        pltpu.sync_copy(seg_hbm.at[pl.ds(r0, _SUB)], segbuf)
        pltpu.sync_copy(e_hbm.at[pl.ds(r0, _SUB)], ebuf)
        pltpu.sync_copy(y_hbm.at[pl.ds(r0, _SUB)], ybuf)
        pltpu.sync_copy(ybuf, nsh.at[segbuf], add=True)
        pltpu.sync_copy(ebuf, dsh.at[segbuf], add=True)
        return 0

    lax.fori_loop(0, _NSUB, sub, 0)
    plsc.subcore_barrier()

    baseL = pl.multiple_of(s * _SLICE, 128)
    pltpu.sync_copy(nsh.at[pl.ds(base, _SLICE)],
                    numer_out.at[c, pl.ds(base, _SLICE)])
    pltpu.sync_copy(dsh.at[pl.ds(base, _SLICE)],
                    den_out.at[c, 0, pl.ds(baseL, _SLICE)])


def _scatter(y, e1, seg1):
    mesh = plsc.VectorSubcoreMesh(core_axis_name="c", subcore_axis_name="s")
    f = pl.kernel(
        _scatter_body,
        out_type=(
            jax.ShapeDtypeStruct((2, _SP, _D), jnp.float32),
            jax.ShapeDtypeStruct((2, 1, _SP), jnp.float32),
        ),
        mesh=mesh,
        scratch_types=[
            pltpu.VMEM_SHARED((_SP, _D), jnp.float32),
            pltpu.VMEM_SHARED((_SP,), jnp.float32),
            pltpu.VMEM((_SUB, _D), jnp.float32),
            pltpu.VMEM((_SUB,), jnp.int32),
            pltpu.VMEM((_SUB,), jnp.float32),
            pltpu.VMEM((_SLICE,), jnp.float32),
        ],
    )
    return f(y, e1, seg1)


_BS = 640


def _combine_body(n_ref, d_ref, o_ref):
    n = n_ref[0] + n_ref[1]
    d = (d_ref[0, 0] + d_ref[1, 0]).reshape(_BS, 1)
    o_ref[...] = jnp.where(d > 0.0, n / jnp.where(d > 0.0, d, 1.0), 0.0)


def _combine(numer, den):
    return pl.pallas_call(
        _combine_body,
        grid=(_SP // _BS,),
        in_specs=[
            pl.BlockSpec((2, _BS, _D), lambda i: (0, i, 0)),
            pl.BlockSpec((2, 1, _BS), lambda i: (0, 0, i)),
        ],
        out_specs=pl.BlockSpec((_BS, _D), lambda i: (i, 0)),
        out_shape=jax.ShapeDtypeStruct((_SP, _D), jnp.float32),
    )(numer, den)


def kernel(x, segment_ids, num_segments, W, b):
    y, e = _weights(x, W, b.reshape(1, 1))
    numer, den = _scatter(y, e[:, 0], segment_ids)
    return _combine(numer, den)[:_S]


# speedup vs baseline: 9.0140x; 1.2218x over previous
import jax
import jax.numpy as jnp
from jax import lax
from jax.experimental import pallas as pl
from jax.experimental.pallas import tpu as pltpu
from jax.experimental.pallas import tpu_sc as plsc

_N = 320000
_D = 128
_S = 10000

_BN = 2000
_RPT = _N // 32
_SUB = 80
_NSUB = _RPT // _SUB
_SP = 10240
_SLICE = _SP // 16


def _weights_body(x_ref, w_ref, b_ref, y_ref, e_ref):
    xb = x_ref[...]
    logit = jnp.dot(xb, w_ref[...], preferred_element_type=jnp.float32)
    e = jnp.exp(logit + b_ref[0, 0])
    y_ref[...] = xb * e
    e_ref[...] = e


def _weights(x, W, b2):
    return pl.pallas_call(
        _weights_body,
        grid=(_N // _BN,),
        in_specs=[
            pl.BlockSpec((_BN, _D), lambda i: (i, 0)),
            pl.BlockSpec((_D, 1), lambda i: (0, 0)),
            pl.BlockSpec((1, 1), lambda i: (0, 0)),
        ],
        out_specs=[
            pl.BlockSpec((_BN, _D), lambda i: (i, 0)),
            pl.BlockSpec((_BN, 1), lambda i: (i, 0)),
        ],
        out_shape=[
            jax.ShapeDtypeStruct((_N, _D), jnp.float32),
            jax.ShapeDtypeStruct((_N, 1), jnp.float32),
        ],
    )(x, W, b2)


def _scatter_body(y_hbm, e_hbm, seg_hbm, numer_out, den_out,
                  nsh, dsh, ybuf, segbuf, ebuf, zv,
                  gs1, gs2, gs3, ss1, ss2):
    c = lax.axis_index("c")
    s = lax.axis_index("s")
    tile = c * 16 + s
    row0 = tile * _RPT

    z16 = jnp.zeros((16,), jnp.float32)

    def zy(i, _):
        for j in range(8):
            ybuf[i, pl.ds(16 * j, 16)] = z16
        return 0

    lax.fori_loop(0, _SUB, zy, 0)

    for q in range(_SLICE // 16):
        zv[pl.ds(16 * q, 16)] = z16

    base = pl.multiple_of(s * _SLICE, 8)
    for j in range(_SLICE // _SUB):
        pltpu.sync_copy(ybuf, nsh.at[pl.ds(base + _SUB * j, _SUB)])
    pltpu.sync_copy(zv, dsh.at[pl.ds(base, _SLICE)])
    plsc.subcore_barrier()

    def sub(j, _):
        r0 = pl.multiple_of(row0 + j * _SUB, 8)
        g1 = pltpu.async_copy(seg_hbm.at[pl.ds(r0, _SUB)], segbuf, gs1)
        g2 = pltpu.async_copy(e_hbm.at[pl.ds(r0, _SUB)], ebuf, gs2)
        g3 = pltpu.async_copy(y_hbm.at[pl.ds(r0, _SUB)], ybuf, gs3)
        g1.wait()
        g2.wait()
        g3.wait()
        s1 = pltpu.async_copy(ybuf, nsh.at[segbuf], ss1, add=True)
        s2 = pltpu.async_copy(ebuf, dsh.at[segbuf], ss2, add=True)
        s1.wait()
        s2.wait()
        return 0

    lax.fori_loop(0, _NSUB, sub, 0)
    plsc.subcore_barrier()

    baseL = pl.multiple_of(s * _SLICE, 128)
    pltpu.sync_copy(nsh.at[pl.ds(base, _SLICE)],
                    numer_out.at[c, pl.ds(base, _SLICE)])
    pltpu.sync_copy(dsh.at[pl.ds(base, _SLICE)],
                    den_out.at[c, 0, pl.ds(baseL, _SLICE)])


def _scatter(y, e1, seg1):
    mesh = plsc.VectorSubcoreMesh(core_axis_name="c", subcore_axis_name="s")
    f = pl.kernel(
        _scatter_body,
        out_type=(
            jax.ShapeDtypeStruct((2, _SP, _D), jnp.float32),
            jax.ShapeDtypeStruct((2, 1, _SP), jnp.float32),
        ),
        mesh=mesh,
        scratch_types=[
            pltpu.VMEM_SHARED((_SP, _D), jnp.float32),
            pltpu.VMEM_SHARED((_SP,), jnp.float32),
            pltpu.VMEM((_SUB, _D), jnp.float32),
            pltpu.VMEM((_SUB,), jnp.int32),
            pltpu.VMEM((_SUB,), jnp.float32),
            pltpu.VMEM((_SLICE,), jnp.float32),
            pltpu.SemaphoreType.DMA,
            pltpu.SemaphoreType.DMA,
            pltpu.SemaphoreType.DMA,
            pltpu.SemaphoreType.DMA,
            pltpu.SemaphoreType.DMA,
        ],
    )
    return f(y, e1, seg1)


_BS = 640


def _combine_body(n_ref, d_ref, o_ref):
    n = n_ref[0] + n_ref[1]
    d = (d_ref[0, 0] + d_ref[1, 0]).reshape(_BS, 1)
    o_ref[...] = jnp.where(d > 0.0, n / jnp.where(d > 0.0, d, 1.0), 0.0)


def _combine(numer, den):
    return pl.pallas_call(
        _combine_body,
        grid=(_SP // _BS,),
        in_specs=[
            pl.BlockSpec((2, _BS, _D), lambda i: (0, i, 0)),
            pl.BlockSpec((2, 1, _BS), lambda i: (0, 0, i)),
        ],
        out_specs=pl.BlockSpec((_BS, _D), lambda i: (i, 0)),
        out_shape=jax.ShapeDtypeStruct((_SP, _D), jnp.float32),
    )(numer, den)


def kernel(x, segment_ids, num_segments, W, b):
    y, e = _weights(x, W, b.reshape(1, 1))
    numer, den = _scatter(y, e[:, 0], segment_ids)
    return _combine(numer, den)[:_S]


# speedup vs baseline: 10.0483x; 1.1147x over previous
import jax
import jax.numpy as jnp
from jax import lax
from jax.experimental import pallas as pl
from jax.experimental.pallas import tpu as pltpu
from jax.experimental.pallas import tpu_sc as plsc

_N = 320000
_D = 128
_S = 10000

_BN = 2000
_RPT = _N // 32
_SUB = 80
_NSUB = _RPT // _SUB
_SP = 10240
_SLICE = _SP // 16


def _weights_body(x_ref, w_ref, b_ref, y_ref, e_ref):
    xb = x_ref[...]
    logit = jnp.dot(xb, w_ref[...], preferred_element_type=jnp.float32)
    e = jnp.exp(logit + b_ref[0, 0])
    y_ref[...] = xb * e
    e_ref[...] = e


def _weights(x, W, b2):
    return pl.pallas_call(
        _weights_body,
        grid=(_N // _BN,),
        in_specs=[
            pl.BlockSpec((_BN, _D), lambda i: (i, 0)),
            pl.BlockSpec((_D, 1), lambda i: (0, 0)),
            pl.BlockSpec((1, 1), lambda i: (0, 0)),
        ],
        out_specs=[
            pl.BlockSpec((_BN, _D), lambda i: (i, 0)),
            pl.BlockSpec((_BN, 1), lambda i: (i, 0)),
        ],
        out_shape=[
            jax.ShapeDtypeStruct((_N, _D), jnp.float32),
            jax.ShapeDtypeStruct((_N, 1), jnp.float32),
        ],
    )(x, W, b2)


def _scatter_body(y_hbm, e_hbm, seg_hbm, numer_out, den_out,
                  nsh, dsh, ybuf, ybuf2, segbuf, segbuf2, ebuf, ebuf2, zv,
                  gs1, gs2, gs3, gt1, gt2, gt3, ss1, ss2, st1, st2):
    c = lax.axis_index("c")
    s = lax.axis_index("s")
    tile = c * 16 + s
    row0 = tile * _RPT

    z16 = jnp.zeros((16,), jnp.float32)

    def zy(i, _):
        for j in range(8):
            ybuf[i, pl.ds(16 * j, 16)] = z16
        return 0

    lax.fori_loop(0, _SUB, zy, 0)

    for q in range(_SLICE // 16):
        zv[pl.ds(16 * q, 16)] = z16

    base = pl.multiple_of(s * _SLICE, 8)
    for j in range(_SLICE // _SUB):
        pltpu.sync_copy(ybuf, nsh.at[pl.ds(base + _SUB * j, _SUB)])
    pltpu.sync_copy(zv, dsh.at[pl.ds(base, _SLICE)])
    plsc.subcore_barrier()

    bufs = ((segbuf, ebuf, ybuf, gs1, gs2, gs3, ss1, ss2),
            (segbuf2, ebuf2, ybuf2, gt1, gt2, gt3, st1, st2))

    def g_issue(r0, p):
        sb, eb, yb, a, b, d, _, _ = bufs[p]
        pltpu.async_copy(seg_hbm.at[pl.ds(r0, _SUB)], sb, a)
        pltpu.async_copy(e_hbm.at[pl.ds(r0, _SUB)], eb, b)
        pltpu.async_copy(y_hbm.at[pl.ds(r0, _SUB)], yb, d)

    def g_wait(p):
        sb, eb, yb, a, b, d, _, _ = bufs[p]
        pltpu.make_async_copy(seg_hbm.at[pl.ds(0, _SUB)], sb, a).wait()
        pltpu.make_async_copy(e_hbm.at[pl.ds(0, _SUB)], eb, b).wait()
        pltpu.make_async_copy(y_hbm.at[pl.ds(0, _SUB)], yb, d).wait()

    def s_issue(p):
        sb, eb, yb, _, _, _, u, v = bufs[p]
        pltpu.async_copy(yb, nsh.at[sb], u, add=True)
        pltpu.async_copy(eb, dsh.at[sb], v, add=True)

    def s_wait(p):
        sb, eb, yb, _, _, _, u, v = bufs[p]
        pltpu.make_async_copy(yb, nsh.at[sb], u).wait()
        pltpu.make_async_copy(eb, dsh.at[sb], v).wait()

    def rof(j):
        r = jnp.where(j < _NSUB, row0 + j * _SUB, row0)
        return pl.multiple_of(r, 8)

    g_issue(rof(0), 0)
    g_wait(0)
    s_issue(0)
    g_issue(rof(1), 1)

    def pair(t, _):
        for h in range(2):
            j = 2 * t + 1 + h
            p = 1 - h
            g_wait(p)
            s_issue(p)
            s_wait(1 - p)
            g_issue(rof(j + 1), 1 - p)
        return 0

    lax.fori_loop(0, (_NSUB - 1) // 2, pair, 0)
    s_wait(0)
    g_wait(1)
    plsc.subcore_barrier()

    baseL = pl.multiple_of(s * _SLICE, 128)
    pltpu.sync_copy(nsh.at[pl.ds(base, _SLICE)],
                    numer_out.at[c, pl.ds(base, _SLICE)])
    pltpu.sync_copy(dsh.at[pl.ds(base, _SLICE)],
                    den_out.at[c, 0, pl.ds(baseL, _SLICE)])


def _scatter(y, e1, seg1):
    mesh = plsc.VectorSubcoreMesh(core_axis_name="c", subcore_axis_name="s")
    f = pl.kernel(
        _scatter_body,
        out_type=(
            jax.ShapeDtypeStruct((2, _SP, _D), jnp.float32),
            jax.ShapeDtypeStruct((2, 1, _SP), jnp.float32),
        ),
        mesh=mesh,
        scratch_types=[
            pltpu.VMEM_SHARED((_SP, _D), jnp.float32),
            pltpu.VMEM_SHARED((_SP,), jnp.float32),
            pltpu.VMEM((_SUB, _D), jnp.float32),
            pltpu.VMEM((_SUB, _D), jnp.float32),
            pltpu.VMEM((_SUB,), jnp.int32),
            pltpu.VMEM((_SUB,), jnp.int32),
            pltpu.VMEM((_SUB,), jnp.float32),
            pltpu.VMEM((_SUB,), jnp.float32),
            pltpu.VMEM((_SLICE,), jnp.float32),
            pltpu.SemaphoreType.DMA,
            pltpu.SemaphoreType.DMA,
            pltpu.SemaphoreType.DMA,
            pltpu.SemaphoreType.DMA,
            pltpu.SemaphoreType.DMA,
            pltpu.SemaphoreType.DMA,
            pltpu.SemaphoreType.DMA,
            pltpu.SemaphoreType.DMA,
            pltpu.SemaphoreType.DMA,
            pltpu.SemaphoreType.DMA,
        ],
    )
    return f(y, e1, seg1)


_BS = 640


def _combine_body(n_ref, d_ref, o_ref):
    n = n_ref[0] + n_ref[1]
    d = (d_ref[0, 0] + d_ref[1, 0]).reshape(_BS, 1)
    o_ref[...] = jnp.where(d > 0.0, n / jnp.where(d > 0.0, d, 1.0), 0.0)


def _combine(numer, den):
    return pl.pallas_call(
        _combine_body,
        grid=(_SP // _BS,),
        in_specs=[
            pl.BlockSpec((2, _BS, _D), lambda i: (0, i, 0)),
            pl.BlockSpec((2, 1, _BS), lambda i: (0, 0, i)),
        ],
        out_specs=pl.BlockSpec((_BS, _D), lambda i: (i, 0)),
        out_shape=jax.ShapeDtypeStruct((_SP, _D), jnp.float32),
    )(numer, den)


def kernel(x, segment_ids, num_segments, W, b):
    y, e = _weights(x, W, b.reshape(1, 1))
    numer, den = _scatter(y, e[:, 0], segment_ids)
    return _combine(numer, den)[:_S]
